# baseline (speedup 1.0000x reference)
import functools

import jax
import jax.numpy as jnp
from jax import lax
from jax.experimental import pallas as pl
from jax.experimental.pallas import tpu as pltpu
from jax.experimental.pallas import tpu_sc as plsc

N = 10000
N_ACC = 10240
D = 128
DEG_L = 16
NC, NS = 2, 16
NW = NC * NS
CHUNK = 80
NCHUNK = 125
ROWS_Z = N_ACC // NS


def _sc_mesh():
    return plsc.VectorSubcoreMesh(
        core_axis_name="c", subcore_axis_name="s", num_cores=NC, num_subcores=NS
    )


def _sc_degree(dst3):
    @functools.partial(
        pl.kernel,
        mesh=_sc_mesh(),
        out_type=jax.ShapeDtypeStruct((NC, N_ACC), jnp.float32),
        scratch_types=[
            pltpu.VMEM((NCHUNK, CHUNK), jnp.int32),
            pltpu.VMEM((128,), jnp.float32),
            pltpu.VMEM((ROWS_Z,), jnp.float32),
            pltpu.VMEM_SHARED((N_ACC,), jnp.float32),
            pltpu.SemaphoreType.DMA,
        ],
    )
    def k(dst_hbm, out_hbm, dst_v, ones_v, zeros_v, acc_sh, sem):
        c = lax.axis_index("c")
        s = lax.axis_index("s")
        wid = c * NS + s
        pltpu.sync_copy(dst_hbm.at[wid], dst_v)
        for j in range(128 // 16):
            ones_v[pl.ds(j * 16, 16)] = jnp.full((16,), 1.0, jnp.float32)
        for j in range(ROWS_Z // 16):
            zeros_v[pl.ds(j * 16, 16)] = jnp.zeros((16,), jnp.float32)
        pltpu.sync_copy(zeros_v, acc_sh.at[pl.ds(s * ROWS_Z, ROWS_Z)])
        plsc.subcore_barrier()

        ones_c = ones_v.at[pl.ds(0, CHUNK)]

        def body(i, carry):
            for j in range(5):
                pltpu.async_copy(
                    ones_c, acc_sh.at[dst_v.at[i * 5 + j]], sem, add=True
                )
            for j in range(5):
                pltpu.make_async_copy(
                    ones_c, acc_sh.at[dst_v.at[i * 5 + j]], sem
                ).wait()
            return carry

        lax.fori_loop(0, NCHUNK // 5, body, 0)
        plsc.subcore_barrier()
        pltpu.sync_copy(
            acc_sh.at[pl.ds(s * ROWS_Z, ROWS_Z)],
            out_hbm.at[c, pl.ds(s * ROWS_Z, ROWS_Z)],
        )

    return k(dst3)


def _sc_scatter(g, src2, dst2):
    @functools.partial(
        pl.kernel,
        mesh=_sc_mesh(),
        out_type=jax.ShapeDtypeStruct((NC, N_ACC, D), jnp.float32),
        scratch_types=[
            [pltpu.VMEM((CHUNK,), jnp.int32)] * 3,
            [pltpu.VMEM((CHUNK,), jnp.int32)] * 3,
            [pltpu.VMEM((CHUNK, D), jnp.float32)] * 3,
            pltpu.VMEM_SHARED((N_ACC, D), jnp.float32),
            [pltpu.SemaphoreType.DMA] * 3,
            [pltpu.SemaphoreType.DMA] * 3,
            [pltpu.SemaphoreType.DMA] * 3,
            [pltpu.SemaphoreType.DMA] * 3,
        ],
    )
    def k(g_hbm, src_hbm, dst_hbm, out_hbm,
          srcb, dstb, rows, acc_sh, is_, id_, gs, ss):
        c = lax.axis_index("c")
        s = lax.axis_index("s")
        wid = c * NS + s
        base = wid * NCHUNK

        def zrow(i, carry):
            for j in range(D // 16):
                rows[0][i, pl.ds(j * 16, 16)] = jnp.zeros((16,), jnp.float32)
            return carry

        lax.fori_loop(0, CHUNK, zrow, 0)
        for t in range(ROWS_Z // CHUNK):
            pltpu.sync_copy(rows[0], acc_sh.at[pl.ds(s * ROWS_Z + t * CHUNK, CHUNK)])
        plsc.subcore_barrier()

        for m in range(3):
            pltpu.async_copy(src_hbm.at[base + m], srcb[m], is_[m])
            pltpu.async_copy(dst_hbm.at[base + m], dstb[m], id_[m])
        for m in range(3):
            pltpu.make_async_copy(src_hbm.at[base + m], srcb[m], is_[m]).wait()
            pltpu.async_copy(g_hbm.at[srcb[m]], rows[m], gs[m])

        def body(i, carry):
            k0 = 3 * i
            for m in range(3):
                k = k0 + m
                pltpu.make_async_copy(g_hbm.at[srcb[m]], rows[m], gs[m]).wait()
                pltpu.make_async_copy(dst_hbm.at[base + k], dstb[m], id_[m]).wait()
                pltpu.async_copy(rows[m], acc_sh.at[dstb[m]], ss[m], add=True)

                @pl.when(k + 3 < NCHUNK)
                def _():
                    pltpu.async_copy(src_hbm.at[base + k + 3], srcb[m], is_[m])

            for m in range(3):
                k = k0 + m
                pltpu.make_async_copy(rows[m], acc_sh.at[dstb[m]], ss[m]).wait()

                @pl.when(k + 3 < NCHUNK)
                def _():
                    pltpu.async_copy(dst_hbm.at[base + k + 3], dstb[m], id_[m])
                    pltpu.make_async_copy(
                        src_hbm.at[base + k + 3], srcb[m], is_[m]
                    ).wait()
                    pltpu.async_copy(g_hbm.at[srcb[m]], rows[m], gs[m])

            return carry

        lax.fori_loop(0, NCHUNK // 3, body, 0)
        for m in range(NCHUNK - 3 * (NCHUNK // 3)):
            k = 3 * (NCHUNK // 3) + m
            pltpu.make_async_copy(g_hbm.at[srcb[m]], rows[m], gs[m]).wait()
            pltpu.make_async_copy(dst_hbm.at[base + k], dstb[m], id_[m]).wait()
            pltpu.async_copy(rows[m], acc_sh.at[dstb[m]], ss[m], add=True)
            pltpu.make_async_copy(rows[m], acc_sh.at[dstb[m]], ss[m]).wait()
        plsc.subcore_barrier()
        pltpu.sync_copy(
            acc_sh.at[pl.ds(s * ROWS_Z, ROWS_Z)],
            out_hbm.at[c, pl.ds(s * ROWS_Z, ROWS_Z)],
        )

    return k(g, src2, dst2)


def _tc_prep(x, W_pre, b_pre2, W_conv, deg_b):
    R = 1000
    grid = N // R

    def body(x_ref, wp_ref, bp_ref, wc_ref, degb_ref, ori_ref, g_ref):
        ori = (
            jnp.dot(x_ref[...], wp_ref[...], preferred_element_type=jnp.float32)
            + bp_ref[...]
        )
        ori_ref[...] = ori
        h2 = jnp.dot(ori, wc_ref[...], preferred_element_type=jnp.float32)
        g_ref[...] = lax.rsqrt(degb_ref[...]) * h2

    return pl.pallas_call(
        body,
        grid=(grid,),
        in_specs=[
            pl.BlockSpec((R, D), lambda i: (i, 0)),
            pl.BlockSpec((D, D), lambda i: (0, 0)),
            pl.BlockSpec((1, D), lambda i: (0, 0)),
            pl.BlockSpec((D, D), lambda i: (0, 0)),
            pl.BlockSpec((R, D), lambda i: (i, 0)),
        ],
        out_specs=[
            pl.BlockSpec((R, D), lambda i: (i, 0)),
            pl.BlockSpec((R, D), lambda i: (i, 0)),
        ],
        out_shape=[
            jax.ShapeDtypeStruct((N, D), jnp.float32),
            jax.ShapeDtypeStruct((N, D), jnp.float32),
        ],
    )(x, W_pre, b_pre2, W_conv, deg_b)


def _tc_finish(P, g, deg_b, b_conv2):
    R = 1000
    grid = N // R

    def body(p_ref, g_ref, degb_ref, bc_ref, out_ref):
        tot = p_ref[0] + p_ref[1] + g_ref[...]
        out_ref[...] = jnp.maximum(
            lax.rsqrt(degb_ref[...]) * tot + bc_ref[...], 0.0
        )

    return pl.pallas_call(
        body,
        grid=(grid,),
        in_specs=[
            pl.BlockSpec((NC, R, D), lambda i: (0, i, 0)),
            pl.BlockSpec((R, D), lambda i: (i, 0)),
            pl.BlockSpec((R, D), lambda i: (i, 0)),
            pl.BlockSpec((1, D), lambda i: (0, 0)),
        ],
        out_specs=pl.BlockSpec((R, D), lambda i: (i, 0)),
        out_shape=jax.ShapeDtypeStruct((N, D), jnp.float32),
    )(P, g, deg_b, b_conv2)


def kernel(x, edge_index, W_pre, b_pre, W_conv, b_conv):
    src2 = edge_index[0].reshape(NW * NCHUNK, CHUNK)
    dst2 = edge_index[1].reshape(NW * NCHUNK, CHUNK)
    dst3 = edge_index[1].reshape(NW, NCHUNK, CHUNK)
    deg_parts = _sc_degree(dst3)
    deg = (deg_parts[0] + deg_parts[1] + 1.0)[:N]
    deg_b = jnp.broadcast_to(deg[:, None], (N, D))

    ori, g = _tc_prep(x, W_pre, b_pre[None, :], W_conv, deg_b)
    P = _sc_scatter(g, src2, dst2)
    h = _tc_finish(P, g, deg_b, b_conv[None, :])
    return h, ori


# speedup vs baseline: 1.0321x; 1.0321x over previous
import functools

import jax
import jax.numpy as jnp
from jax import lax
from jax.experimental import pallas as pl
from jax.experimental.pallas import tpu as pltpu
from jax.experimental.pallas import tpu_sc as plsc

N = 10000
N_ACC = 10240
D = 128
DEG_L = 16
NC, NS = 2, 16
NW = NC * NS
CHUNK = 80
NCHUNK = 125
ROWS_Z = N_ACC // NS


def _sc_mesh():
    return plsc.VectorSubcoreMesh(
        core_axis_name="c", subcore_axis_name="s", num_cores=NC, num_subcores=NS
    )


def _sc_degree(dst3):
    @functools.partial(
        pl.kernel,
        mesh=_sc_mesh(),
        out_type=jax.ShapeDtypeStruct((NC, N_ACC), jnp.float32),
        scratch_types=[
            pltpu.VMEM((NCHUNK, CHUNK), jnp.int32),
            pltpu.VMEM((128,), jnp.float32),
            pltpu.VMEM((ROWS_Z,), jnp.float32),
            pltpu.VMEM_SHARED((N_ACC,), jnp.float32),
            pltpu.SemaphoreType.DMA,
        ],
    )
    def k(dst_hbm, out_hbm, dst_v, ones_v, zeros_v, acc_sh, sem):
        c = lax.axis_index("c")
        s = lax.axis_index("s")
        wid = c * NS + s
        pltpu.sync_copy(dst_hbm.at[wid], dst_v)
        for j in range(128 // 16):
            ones_v[pl.ds(j * 16, 16)] = jnp.full((16,), 1.0, jnp.float32)
        for j in range(ROWS_Z // 16):
            zeros_v[pl.ds(j * 16, 16)] = jnp.zeros((16,), jnp.float32)
        pltpu.sync_copy(zeros_v, acc_sh.at[pl.ds(s * ROWS_Z, ROWS_Z)])
        plsc.subcore_barrier()

        ones_c = ones_v.at[pl.ds(0, CHUNK)]

        def body(i, carry):
            for j in range(5):
                pltpu.async_copy(
                    ones_c, acc_sh.at[dst_v.at[i * 5 + j]], sem, add=True
                )
            for j in range(5):
                pltpu.make_async_copy(
                    ones_c, acc_sh.at[dst_v.at[i * 5 + j]], sem
                ).wait()
            return carry

        lax.fori_loop(0, NCHUNK // 5, body, 0)
        plsc.subcore_barrier()
        pltpu.sync_copy(
            acc_sh.at[pl.ds(s * ROWS_Z, ROWS_Z)],
            out_hbm.at[c, pl.ds(s * ROWS_Z, ROWS_Z)],
        )

    return k(dst3)


def _sc_scatter(g, src2, dst2):
    @functools.partial(
        pl.kernel,
        mesh=_sc_mesh(),
        out_type=jax.ShapeDtypeStruct((NC, N_ACC, D), jnp.float32),
        scratch_types=[
            [pltpu.VMEM((CHUNK,), jnp.int32)] * 3,
            [pltpu.VMEM((CHUNK,), jnp.int32)] * 3,
            [pltpu.VMEM((CHUNK, D), jnp.float32)] * 3,
            pltpu.VMEM_SHARED((N_ACC, D), jnp.float32),
            [pltpu.SemaphoreType.DMA] * 3,
            [pltpu.SemaphoreType.DMA] * 3,
            [pltpu.SemaphoreType.DMA] * 3,
            [pltpu.SemaphoreType.DMA] * 3,
            [pltpu.SemaphoreType.DMA] * 3,
        ],
    )
    def k(g_hbm, src_hbm, dst_hbm, out_hbm,
          srcb, dstb, rows, acc_sh, is_, id_, gs, gs2, ss):

        H = CHUNK // 2

        def fire_gather(m):
            pltpu.async_copy(
                g_hbm.at[srcb[m].at[pl.ds(0, H)]], rows[m].at[pl.ds(0, H)], gs[m]
            )
            pltpu.async_copy(
                g_hbm.at[srcb[m].at[pl.ds(H, H)]], rows[m].at[pl.ds(H, H)], gs2[m]
            )

        def wait_gather(m):
            pltpu.make_async_copy(
                g_hbm.at[srcb[m].at[pl.ds(0, H)]], rows[m].at[pl.ds(0, H)], gs[m]
            ).wait()
            pltpu.make_async_copy(
                g_hbm.at[srcb[m].at[pl.ds(H, H)]], rows[m].at[pl.ds(H, H)], gs2[m]
            ).wait()
        c = lax.axis_index("c")
        s = lax.axis_index("s")
        wid = c * NS + s
        base = wid * NCHUNK

        def zrow(i, carry):
            for j in range(D // 16):
                rows[0][i, pl.ds(j * 16, 16)] = jnp.zeros((16,), jnp.float32)
            return carry

        lax.fori_loop(0, CHUNK, zrow, 0)
        for t in range(ROWS_Z // CHUNK):
            pltpu.sync_copy(rows[0], acc_sh.at[pl.ds(s * ROWS_Z + t * CHUNK, CHUNK)])
        plsc.subcore_barrier()

        for m in range(3):
            pltpu.async_copy(src_hbm.at[base + m], srcb[m], is_[m])
            pltpu.async_copy(dst_hbm.at[base + m], dstb[m], id_[m])
        for m in range(3):
            pltpu.make_async_copy(src_hbm.at[base + m], srcb[m], is_[m]).wait()
            fire_gather(m)

        def body(i, carry):
            k0 = 3 * i
            for m in range(3):
                k = k0 + m
                wait_gather(m)
                pltpu.make_async_copy(dst_hbm.at[base + k], dstb[m], id_[m]).wait()
                pltpu.async_copy(rows[m], acc_sh.at[dstb[m]], ss[m], add=True)

                @pl.when(k + 3 < NCHUNK)
                def _():
                    pltpu.async_copy(src_hbm.at[base + k + 3], srcb[m], is_[m])

            for m in range(3):
                k = k0 + m
                pltpu.make_async_copy(rows[m], acc_sh.at[dstb[m]], ss[m]).wait()

                @pl.when(k + 3 < NCHUNK)
                def _():
                    pltpu.async_copy(dst_hbm.at[base + k + 3], dstb[m], id_[m])
                    pltpu.make_async_copy(
                        src_hbm.at[base + k + 3], srcb[m], is_[m]
                    ).wait()
                    fire_gather(m)

            return carry

        lax.fori_loop(0, NCHUNK // 3, body, 0)
        for m in range(NCHUNK - 3 * (NCHUNK // 3)):
            k = 3 * (NCHUNK // 3) + m
            wait_gather(m)
            pltpu.make_async_copy(dst_hbm.at[base + k], dstb[m], id_[m]).wait()
            pltpu.async_copy(rows[m], acc_sh.at[dstb[m]], ss[m], add=True)
            pltpu.make_async_copy(rows[m], acc_sh.at[dstb[m]], ss[m]).wait()
        plsc.subcore_barrier()
        pltpu.sync_copy(
            acc_sh.at[pl.ds(s * ROWS_Z, ROWS_Z)],
            out_hbm.at[c, pl.ds(s * ROWS_Z, ROWS_Z)],
        )

    return k(g, src2, dst2)


def _tc_prep(x, W_pre, b_pre2, W_conv, deg_b):
    R = 1000
    grid = N // R

    def body(x_ref, wp_ref, bp_ref, wc_ref, degb_ref, ori_ref, g_ref):
        ori = (
            jnp.dot(x_ref[...], wp_ref[...], preferred_element_type=jnp.float32)
            + bp_ref[...]
        )
        ori_ref[...] = ori
        h2 = jnp.dot(ori, wc_ref[...], preferred_element_type=jnp.float32)
        g_ref[...] = lax.rsqrt(degb_ref[...]) * h2

    return pl.pallas_call(
        body,
        grid=(grid,),
        in_specs=[
            pl.BlockSpec((R, D), lambda i: (i, 0)),
            pl.BlockSpec((D, D), lambda i: (0, 0)),
            pl.BlockSpec((1, D), lambda i: (0, 0)),
            pl.BlockSpec((D, D), lambda i: (0, 0)),
            pl.BlockSpec((R, D), lambda i: (i, 0)),
        ],
        out_specs=[
            pl.BlockSpec((R, D), lambda i: (i, 0)),
            pl.BlockSpec((R, D), lambda i: (i, 0)),
        ],
        out_shape=[
            jax.ShapeDtypeStruct((N, D), jnp.float32),
            jax.ShapeDtypeStruct((N, D), jnp.float32),
        ],
    )(x, W_pre, b_pre2, W_conv, deg_b)


def _tc_finish(P, g, deg_b, b_conv2):
    R = 1000
    grid = N // R

    def body(p_ref, g_ref, degb_ref, bc_ref, out_ref):
        tot = p_ref[0] + p_ref[1] + g_ref[...]
        out_ref[...] = jnp.maximum(
            lax.rsqrt(degb_ref[...]) * tot + bc_ref[...], 0.0
        )

    return pl.pallas_call(
        body,
        grid=(grid,),
        in_specs=[
            pl.BlockSpec((NC, R, D), lambda i: (0, i, 0)),
            pl.BlockSpec((R, D), lambda i: (i, 0)),
            pl.BlockSpec((R, D), lambda i: (i, 0)),
            pl.BlockSpec((1, D), lambda i: (0, 0)),
        ],
        out_specs=pl.BlockSpec((R, D), lambda i: (i, 0)),
        out_shape=jax.ShapeDtypeStruct((N, D), jnp.float32),
    )(P, g, deg_b, b_conv2)


def kernel(x, edge_index, W_pre, b_pre, W_conv, b_conv):
    src2 = edge_index[0].reshape(NW * NCHUNK, CHUNK)
    dst2 = edge_index[1].reshape(NW * NCHUNK, CHUNK)
    dst3 = edge_index[1].reshape(NW, NCHUNK, CHUNK)
    deg_parts = _sc_degree(dst3)
    deg = (deg_parts[0] + deg_parts[1] + 1.0)[:N]
    deg_b = jnp.broadcast_to(deg[:, None], (N, D))

    ori, g = _tc_prep(x, W_pre, b_pre[None, :], W_conv, deg_b)
    P = _sc_scatter(g, src2, dst2)
    h = _tc_finish(P, g, deg_b, b_conv[None, :])
    return h, ori


# speedup vs baseline: 1.0415x; 1.0091x over previous
import functools

import jax
import jax.numpy as jnp
from jax import lax
from jax.experimental import pallas as pl
from jax.experimental.pallas import tpu as pltpu
from jax.experimental.pallas import tpu_sc as plsc

N = 10000
N_ACC = 10240
D = 128
DEG_L = 16
NC, NS = 2, 16
NW = NC * NS
CHUNK = 80
NCHUNK = 125
ROWS_Z = N_ACC // NS


def _sc_mesh():
    return plsc.VectorSubcoreMesh(
        core_axis_name="c", subcore_axis_name="s", num_cores=NC, num_subcores=NS
    )


def _sc_degree(dst3):
    @functools.partial(
        pl.kernel,
        mesh=_sc_mesh(),
        out_type=jax.ShapeDtypeStruct((NC, N_ACC), jnp.float32),
        scratch_types=[
            pltpu.VMEM((NCHUNK, CHUNK), jnp.int32),
            pltpu.VMEM((128,), jnp.float32),
            pltpu.VMEM((ROWS_Z,), jnp.float32),
            pltpu.VMEM_SHARED((N_ACC,), jnp.float32),
            pltpu.SemaphoreType.DMA,
        ],
    )
    def k(dst_hbm, out_hbm, dst_v, ones_v, zeros_v, acc_sh, sem):
        c = lax.axis_index("c")
        s = lax.axis_index("s")
        wid = c * NS + s
        pltpu.sync_copy(dst_hbm.at[wid], dst_v)
        for j in range(128 // 16):
            ones_v[pl.ds(j * 16, 16)] = jnp.full((16,), 1.0, jnp.float32)
        for j in range(ROWS_Z // 16):
            zeros_v[pl.ds(j * 16, 16)] = jnp.zeros((16,), jnp.float32)
        pltpu.sync_copy(zeros_v, acc_sh.at[pl.ds(s * ROWS_Z, ROWS_Z)])
        plsc.subcore_barrier()

        ones_c = ones_v.at[pl.ds(0, CHUNK)]

        def body(i, carry):
            for j in range(5):
                pltpu.async_copy(
                    ones_c, acc_sh.at[dst_v.at[i * 5 + j]], sem, add=True
                )
            for j in range(5):
                pltpu.make_async_copy(
                    ones_c, acc_sh.at[dst_v.at[i * 5 + j]], sem
                ).wait()
            return carry

        lax.fori_loop(0, NCHUNK // 5, body, 0)
        plsc.subcore_barrier()
        pltpu.sync_copy(
            acc_sh.at[pl.ds(s * ROWS_Z, ROWS_Z)],
            out_hbm.at[c, pl.ds(s * ROWS_Z, ROWS_Z)],
        )

    return k(dst3)


def _sc_scatter(g, src2, dst2):
    @functools.partial(
        pl.kernel,
        mesh=_sc_mesh(),
        out_type=jax.ShapeDtypeStruct((NC, N_ACC, D), jnp.float32),
        scratch_types=[
            [pltpu.VMEM((CHUNK,), jnp.int32)] * 3,
            [pltpu.VMEM((CHUNK,), jnp.int32)] * 3,
            [pltpu.VMEM((CHUNK, D), jnp.float32)] * 3,
            pltpu.VMEM_SHARED((N_ACC, D), jnp.float32),
            [pltpu.SemaphoreType.DMA] * 3,
            [pltpu.SemaphoreType.DMA] * 3,
            [pltpu.SemaphoreType.DMA] * 3,
            [pltpu.SemaphoreType.DMA] * 3,
            [pltpu.SemaphoreType.DMA] * 3,
        ],
    )
    def k(g_hbm, src_hbm, dst_hbm, out_hbm,
          srcb, dstb, rows, acc_sh, is_, id_, gs, gs2, ss):

        H = CHUNK // 2

        def fire_gather(m):
            pltpu.async_copy(
                g_hbm.at[srcb[m].at[pl.ds(0, H)]], rows[m].at[pl.ds(0, H)], gs[m]
            )
            pltpu.async_copy(
                g_hbm.at[srcb[m].at[pl.ds(H, H)]], rows[m].at[pl.ds(H, H)], gs2[m]
            )

        def wait_gather(m):
            pltpu.make_async_copy(
                g_hbm.at[srcb[m].at[pl.ds(0, H)]], rows[m].at[pl.ds(0, H)], gs[m]
            ).wait()
            pltpu.make_async_copy(
                g_hbm.at[srcb[m].at[pl.ds(H, H)]], rows[m].at[pl.ds(H, H)], gs2[m]
            ).wait()
        c = lax.axis_index("c")
        s = lax.axis_index("s")
        wid = c * NS + s
        base = wid * NCHUNK

        def zrow(i, carry):
            for j in range(D // 16):
                rows[0][i, pl.ds(j * 16, 16)] = jnp.zeros((16,), jnp.float32)
            return carry

        lax.fori_loop(0, CHUNK, zrow, 0)
        for t in range(ROWS_Z // CHUNK):
            pltpu.sync_copy(rows[0], acc_sh.at[pl.ds(s * ROWS_Z + t * CHUNK, CHUNK)])
        plsc.subcore_barrier()

        for m in range(3):
            pltpu.async_copy(src_hbm.at[base + m], srcb[m], is_[m])
            pltpu.async_copy(dst_hbm.at[base + m], dstb[m], id_[m])
        for m in range(3):
            pltpu.make_async_copy(src_hbm.at[base + m], srcb[m], is_[m]).wait()
            fire_gather(m)

        def body(i, carry):
            k0 = 3 * i
            for m in range(3):
                k = k0 + m
                wait_gather(m)
                pltpu.make_async_copy(dst_hbm.at[base + k], dstb[m], id_[m]).wait()
                pltpu.async_copy(rows[m], acc_sh.at[dstb[m]], ss[m], add=True)

                @pl.when(k + 3 < NCHUNK)
                def _():
                    pltpu.async_copy(src_hbm.at[base + k + 3], srcb[m], is_[m])

            for m in range(3):
                k = k0 + m
                pltpu.make_async_copy(rows[m], acc_sh.at[dstb[m]], ss[m]).wait()

                @pl.when(k + 3 < NCHUNK)
                def _():
                    pltpu.async_copy(dst_hbm.at[base + k + 3], dstb[m], id_[m])
                    pltpu.make_async_copy(
                        src_hbm.at[base + k + 3], srcb[m], is_[m]
                    ).wait()
                    fire_gather(m)

            return carry

        lax.fori_loop(0, NCHUNK // 3, body, 0)
        for m in range(NCHUNK - 3 * (NCHUNK // 3)):
            k = 3 * (NCHUNK // 3) + m
            wait_gather(m)
            pltpu.make_async_copy(dst_hbm.at[base + k], dstb[m], id_[m]).wait()
            pltpu.async_copy(rows[m], acc_sh.at[dstb[m]], ss[m], add=True)
            pltpu.make_async_copy(rows[m], acc_sh.at[dstb[m]], ss[m]).wait()
        plsc.subcore_barrier()
        pltpu.sync_copy(
            acc_sh.at[pl.ds(s * ROWS_Z, ROWS_Z)],
            out_hbm.at[c, pl.ds(s * ROWS_Z, ROWS_Z)],
        )

    return k(g, src2, dst2)


def _tc_prep(x, W_pre, b_pre2, W_conv, deg_b):
    R = 1000
    grid = N // R

    def body(x_ref, wp_ref, bp_ref, wc_ref, degb_ref, ori_ref, g_ref):
        ori = (
            jnp.dot(x_ref[...], wp_ref[...], preferred_element_type=jnp.float32)
            + bp_ref[...]
        )
        ori_ref[...] = ori
        h2 = jnp.dot(ori, wc_ref[...], preferred_element_type=jnp.float32)
        dis = lax.rsqrt(lax.broadcast_in_dim(degb_ref[...], (R, D), (0, 1)))
        g_ref[...] = dis * h2

    return pl.pallas_call(
        body,
        grid=(grid,),
        in_specs=[
            pl.BlockSpec((R, D), lambda i: (i, 0)),
            pl.BlockSpec((D, D), lambda i: (0, 0)),
            pl.BlockSpec((1, D), lambda i: (0, 0)),
            pl.BlockSpec((D, D), lambda i: (0, 0)),
            pl.BlockSpec((R, 1), lambda i: (i, 0)),
        ],
        out_specs=[
            pl.BlockSpec((R, D), lambda i: (i, 0)),
            pl.BlockSpec((R, D), lambda i: (i, 0)),
        ],
        out_shape=[
            jax.ShapeDtypeStruct((N, D), jnp.float32),
            jax.ShapeDtypeStruct((N, D), jnp.float32),
        ],
    )(x, W_pre, b_pre2, W_conv, deg_b)


def _tc_finish(P, g, deg_b, b_conv2):
    R = 1000
    grid = N // R

    def body(p_ref, g_ref, degb_ref, bc_ref, out_ref):
        tot = p_ref[0] + p_ref[1] + g_ref[...]
        dis = lax.rsqrt(lax.broadcast_in_dim(degb_ref[...], (R, D), (0, 1)))
        out_ref[...] = jnp.maximum(dis * tot + bc_ref[...], 0.0)

    return pl.pallas_call(
        body,
        grid=(grid,),
        in_specs=[
            pl.BlockSpec((NC, R, D), lambda i: (0, i, 0)),
            pl.BlockSpec((R, D), lambda i: (i, 0)),
            pl.BlockSpec((R, 1), lambda i: (i, 0)),
            pl.BlockSpec((1, D), lambda i: (0, 0)),
        ],
        out_specs=pl.BlockSpec((R, D), lambda i: (i, 0)),
        out_shape=jax.ShapeDtypeStruct((N, D), jnp.float32),
    )(P, g, deg_b, b_conv2)


def kernel(x, edge_index, W_pre, b_pre, W_conv, b_conv):
    src2 = edge_index[0].reshape(NW * NCHUNK, CHUNK)
    dst2 = edge_index[1].reshape(NW * NCHUNK, CHUNK)
    dst3 = edge_index[1].reshape(NW, NCHUNK, CHUNK)
    deg_parts = _sc_degree(dst3)
    deg_b = (deg_parts[0] + deg_parts[1] + 1.0)[:N, None]

    ori, g = _tc_prep(x, W_pre, b_pre[None, :], W_conv, deg_b)
    P = _sc_scatter(g, src2, dst2)
    h = _tc_finish(P, g, deg_b, b_conv[None, :])
    return h, ori


# speedup vs baseline: 1.0694x; 1.0267x over previous
import functools

import jax
import jax.numpy as jnp
from jax import lax
from jax.experimental import pallas as pl
from jax.experimental.pallas import tpu as pltpu
from jax.experimental.pallas import tpu_sc as plsc

N = 10000
N_ACC = 10240
D = 128
NC, NS = 2, 16
NW = NC * NS
CHUNK = 80
NCHUNK = 125
ROWS_Z = N_ACC // NS


def _sc_mesh():
    return plsc.VectorSubcoreMesh(
        core_axis_name="c", subcore_axis_name="s", num_cores=NC, num_subcores=NS
    )


def _sc_degree(dst3):
    @functools.partial(
        pl.kernel,
        mesh=_sc_mesh(),
        out_type=jax.ShapeDtypeStruct((NC, N_ACC), jnp.float32),
        scratch_types=[
            pltpu.VMEM((NCHUNK, CHUNK), jnp.int32),
            pltpu.VMEM((128,), jnp.float32),
            pltpu.VMEM((ROWS_Z,), jnp.float32),
            pltpu.VMEM_SHARED((N_ACC,), jnp.float32),
            pltpu.SemaphoreType.DMA,
        ],
    )
    def k(dst_hbm, out_hbm, dst_v, ones_v, zeros_v, acc_sh, sem):
        c = lax.axis_index("c")
        s = lax.axis_index("s")
        wid = c * NS + s
        pltpu.sync_copy(dst_hbm.at[wid], dst_v)
        for j in range(128 // 16):
            ones_v[pl.ds(j * 16, 16)] = jnp.full((16,), 1.0, jnp.float32)
        for j in range(ROWS_Z // 16):
            zeros_v[pl.ds(j * 16, 16)] = jnp.zeros((16,), jnp.float32)
        pltpu.sync_copy(zeros_v, acc_sh.at[pl.ds(s * ROWS_Z, ROWS_Z)])
        plsc.subcore_barrier()

        ones_c = ones_v.at[pl.ds(0, CHUNK)]

        def body(i, carry):
            for j in range(5):
                pltpu.async_copy(
                    ones_c, acc_sh.at[dst_v.at[i * 5 + j]], sem, add=True
                )
            for j in range(5):
                pltpu.make_async_copy(
                    ones_c, acc_sh.at[dst_v.at[i * 5 + j]], sem
                ).wait()
            return carry

        lax.fori_loop(0, NCHUNK // 5, body, 0)
        plsc.subcore_barrier()
        pltpu.sync_copy(
            acc_sh.at[pl.ds(s * ROWS_Z, ROWS_Z)],
            out_hbm.at[c, pl.ds(s * ROWS_Z, ROWS_Z)],
        )

    return k(dst3)


def _sc_scatter(g, src2, dst2):
    @functools.partial(
        pl.kernel,
        mesh=_sc_mesh(),
        out_type=jax.ShapeDtypeStruct((NC, N_ACC, D), jnp.float32),
        scratch_types=[
            [pltpu.VMEM((CHUNK,), jnp.int32)] * 3,
            [pltpu.VMEM((CHUNK,), jnp.int32)] * 3,
            [pltpu.VMEM((CHUNK, D), jnp.float32)] * 3,
            pltpu.VMEM_SHARED((N_ACC, D), jnp.float32),
            [pltpu.SemaphoreType.DMA] * 3,
            [pltpu.SemaphoreType.DMA] * 3,
            [pltpu.SemaphoreType.DMA] * 3,
            [pltpu.SemaphoreType.DMA] * 3,
            [pltpu.SemaphoreType.DMA] * 3,
        ],
    )
    def k(g_hbm, src_hbm, dst_hbm, out_hbm,
          srcb, dstb, rows, acc_sh, is_, id_, gs, gs2, ss):

        H = CHUNK // 2

        def fire_gather(m):
            pltpu.async_copy(
                g_hbm.at[srcb[m].at[pl.ds(0, H)]], rows[m].at[pl.ds(0, H)], gs[m]
            )
            pltpu.async_copy(
                g_hbm.at[srcb[m].at[pl.ds(H, H)]], rows[m].at[pl.ds(H, H)], gs2[m]
            )

        def wait_gather(m):
            pltpu.make_async_copy(
                g_hbm.at[srcb[m].at[pl.ds(0, H)]], rows[m].at[pl.ds(0, H)], gs[m]
            ).wait()
            pltpu.make_async_copy(
                g_hbm.at[srcb[m].at[pl.ds(H, H)]], rows[m].at[pl.ds(H, H)], gs2[m]
            ).wait()
        c = lax.axis_index("c")
        s = lax.axis_index("s")
        wid = c * NS + s
        base = wid * NCHUNK

        def zrow(i, carry):
            for j in range(D // 16):
                rows[0][i, pl.ds(j * 16, 16)] = jnp.zeros((16,), jnp.float32)
            return carry

        lax.fori_loop(0, CHUNK, zrow, 0)
        for t in range(ROWS_Z // CHUNK):
            pltpu.sync_copy(rows[0], acc_sh.at[pl.ds(s * ROWS_Z + t * CHUNK, CHUNK)])
        plsc.subcore_barrier()

        for m in range(3):
            pltpu.async_copy(src_hbm.at[base + m], srcb[m], is_[m])
            pltpu.async_copy(dst_hbm.at[base + m], dstb[m], id_[m])
        for m in range(3):
            pltpu.make_async_copy(src_hbm.at[base + m], srcb[m], is_[m]).wait()
            fire_gather(m)

        def body(i, carry):
            k0 = 3 * i
            for m in range(3):
                k = k0 + m
                wait_gather(m)
                pltpu.make_async_copy(dst_hbm.at[base + k], dstb[m], id_[m]).wait()
                pltpu.async_copy(rows[m], acc_sh.at[dstb[m]], ss[m], add=True)

                @pl.when(k + 3 < NCHUNK)
                def _():
                    pltpu.async_copy(src_hbm.at[base + k + 3], srcb[m], is_[m])

            for m in range(3):
                k = k0 + m
                pltpu.make_async_copy(rows[m], acc_sh.at[dstb[m]], ss[m]).wait()

                @pl.when(k + 3 < NCHUNK)
                def _():
                    pltpu.async_copy(dst_hbm.at[base + k + 3], dstb[m], id_[m])
                    pltpu.make_async_copy(
                        src_hbm.at[base + k + 3], srcb[m], is_[m]
                    ).wait()
                    fire_gather(m)

            return carry

        lax.fori_loop(0, NCHUNK // 3, body, 0)
        for m in range(NCHUNK - 3 * (NCHUNK // 3)):
            k = 3 * (NCHUNK // 3) + m
            wait_gather(m)
            pltpu.make_async_copy(dst_hbm.at[base + k], dstb[m], id_[m]).wait()
            pltpu.async_copy(rows[m], acc_sh.at[dstb[m]], ss[m], add=True)
            pltpu.make_async_copy(rows[m], acc_sh.at[dstb[m]], ss[m]).wait()
        plsc.subcore_barrier()
        pltpu.sync_copy(
            acc_sh.at[pl.ds(s * ROWS_Z, ROWS_Z)],
            out_hbm.at[c, pl.ds(s * ROWS_Z, ROWS_Z)],
        )

    return k(g, src2, dst2)


def _tc_prep(x, W_pre, b_pre2, W_conv, deg_b):
    R = 2000
    grid = N // R

    def body(x_ref, wp_ref, bp_ref, wc_ref, degb_ref, ori_ref, g_ref):
        ori = (
            jnp.dot(x_ref[...], wp_ref[...], preferred_element_type=jnp.float32)
            + bp_ref[...]
        )
        ori_ref[...] = ori
        h2 = jnp.dot(ori, wc_ref[...], preferred_element_type=jnp.float32)
        dis = lax.rsqrt(lax.broadcast_in_dim(degb_ref[...], (R, D), (0, 1)))
        g_ref[...] = dis * h2

    return pl.pallas_call(
        body,
        grid=(grid,),
        in_specs=[
            pl.BlockSpec((R, D), lambda i: (i, 0)),
            pl.BlockSpec((D, D), lambda i: (0, 0)),
            pl.BlockSpec((1, D), lambda i: (0, 0)),
            pl.BlockSpec((D, D), lambda i: (0, 0)),
            pl.BlockSpec((R, 1), lambda i: (i, 0)),
        ],
        out_specs=[
            pl.BlockSpec((R, D), lambda i: (i, 0)),
            pl.BlockSpec((R, D), lambda i: (i, 0)),
        ],
        out_shape=[
            jax.ShapeDtypeStruct((N, D), jnp.float32),
            jax.ShapeDtypeStruct((N, D), jnp.float32),
        ],
    )(x, W_pre, b_pre2, W_conv, deg_b)


def _tc_finish(P, g, deg_b, b_conv2):
    R = 2000
    grid = N // R

    def body(p_ref, g_ref, degb_ref, bc_ref, out_ref):
        tot = p_ref[0] + p_ref[1] + g_ref[...]
        dis = lax.rsqrt(lax.broadcast_in_dim(degb_ref[...], (R, D), (0, 1)))
        out_ref[...] = jnp.maximum(dis * tot + bc_ref[...], 0.0)

    return pl.pallas_call(
        body,
        grid=(grid,),
        in_specs=[
            pl.BlockSpec((NC, R, D), lambda i: (0, i, 0)),
            pl.BlockSpec((R, D), lambda i: (i, 0)),
            pl.BlockSpec((R, 1), lambda i: (i, 0)),
            pl.BlockSpec((1, D), lambda i: (0, 0)),
        ],
        out_specs=pl.BlockSpec((R, D), lambda i: (i, 0)),
        out_shape=jax.ShapeDtypeStruct((N, D), jnp.float32),
    )(P, g, deg_b, b_conv2)


def kernel(x, edge_index, W_pre, b_pre, W_conv, b_conv):
    src2 = edge_index[0].reshape(NW * NCHUNK, CHUNK)
    dst2 = edge_index[1].reshape(NW * NCHUNK, CHUNK)
    dst3 = edge_index[1].reshape(NW, NCHUNK, CHUNK)
    deg_parts = _sc_degree(dst3)
    deg_b = (deg_parts[0] + deg_parts[1] + 1.0)[:N, None]

    ori, g = _tc_prep(x, W_pre, b_pre[None, :], W_conv, deg_b)
    P = _sc_scatter(g, src2, dst2)
    h = _tc_finish(P, g, deg_b, b_conv[None, :])
    return h, ori
